# Initial kernel scaffold; baseline (speedup 1.0000x reference)
import jax
import jax.numpy as jnp
from jax.experimental import pallas as pl


def kernel(points_2d, camera_indices, point_indices, pose, points_3d):
    raise NotImplementedError("write your pallas kernel here")


# speedup vs baseline: 2.8990x; 2.8990x over previous
import functools

import jax
import jax.numpy as jnp
from jax import lax
from jax.experimental import pallas as pl
from jax.experimental.pallas import tpu as pltpu
from jax.experimental.pallas import tpu_sc as plsc

N_IMGS = 2000
N_PTS = 500000
N_OBS = 2000000

NC = 2
NS = 16
NW = NC * NS
L = 16

CHUNK = 3200
BLK = 128
NBLK = CHUNK // BLK
NCHUNKS = N_OBS // CHUNK
KMAX = -(-NCHUNKS // NW)


def _body(p2d_hbm, ci_hbm, pi_hbm, pose_hbm, pts3_hbm, out_hbm,
          pose_v, ci_v, pi_v, p2d_v, pts_v, out_v, sem):
    wid = lax.axis_index("s") * NC + lax.axis_index("c")
    pltpu.sync_copy(pose_hbm, pose_v)

    iota = lax.iota(jnp.int32, L)
    zero = jnp.zeros((L,), jnp.int32)
    one = jnp.full((L,), 1, jnp.int32)
    two = jnp.full((L,), 2, jnp.int32)
    ccol = [jnp.full((L,), j, jnp.int32) for j in range(10)]

    def do_chunk(k, _):
        c = k * NW + wid

        @pl.when(c < NCHUNKS)
        def _():
            base = c * CHUNK
            pltpu.sync_copy(ci_hbm.at[pl.ds(base, CHUNK)], ci_v)
            pltpu.sync_copy(pi_hbm.at[c], pi_v)
            pltpu.sync_copy(p2d_hbm.at[pl.ds(base, CHUNK)], p2d_v)
            copies = [
                pltpu.async_copy(pts3_hbm.at[pi_v.at[j]],
                                 pts_v.at[pl.ds(j * BLK, BLK)], sem)
                for j in range(NBLK)
            ]
            for cp in copies:
                cp.wait()

            def grp(g, _):
                o = g * L + iota
                ci = ci_v[pl.ds(g * L, L)]
                cam = [plsc.load_gather(pose_v, [ci, ccol[j]])
                       for j in range(10)]
                tx, ty, tz, qx, qy, qz, qw, fo, k1, k2 = cam
                px = plsc.load_gather(pts_v, [o, zero])
                py = plsc.load_gather(pts_v, [o, one])
                pz = plsc.load_gather(pts_v, [o, two])
                u2d = plsc.load_gather(p2d_v, [o, zero])
                v2d = plsc.load_gather(p2d_v, [o, one])

                nq = qx * qx + qy * qy + qz * qz + qw * qw
                c1x = qy * pz - qz * py
                c1y = qz * px - qx * pz
                c1z = qx * py - qy * px
                c2x = qy * c1z - qz * c1y
                c2y = qz * c1x - qx * c1z
                c2z = qx * c1y - qy * c1x
                s = 2.0 / nq
                rx = px + s * (qw * c1x + c2x) + tx
                ry = py + s * (qw * c1y + c2y) + ty
                rz = pz + s * (qw * c1z + c2z) + tz
                u = rx / rz
                v = ry / rz
                nr = u * u + v * v
                rad = 1.0 + k1 * nr + k2 * nr * nr
                rf = rad * fo
                plsc.store_scatter(out_v, [o, zero], u * rf - u2d)
                plsc.store_scatter(out_v, [o, one], v * rf - v2d)
                return 0

            lax.fori_loop(0, CHUNK // L, grp, 0)
            pltpu.sync_copy(out_v, out_hbm.at[pl.ds(base, CHUNK)])

        return 0

    lax.fori_loop(0, KMAX, do_chunk, 0)


def kernel(points_2d, camera_indices, point_indices, pose, points_3d):
    pi3 = point_indices.reshape(NCHUNKS, NBLK, BLK)
    mesh = plsc.VectorSubcoreMesh(core_axis_name="c", subcore_axis_name="s")
    f = pl.kernel(
        _body,
        mesh=mesh,
        compiler_params=pltpu.CompilerParams(
            needs_layout_passes=False, use_tc_tiling_on_sc=False),
        out_type=jax.ShapeDtypeStruct((N_OBS, 2), jnp.float32),
        scratch_types=[
            pltpu.VMEM((N_IMGS, 10), jnp.float32),
            pltpu.VMEM((CHUNK,), jnp.int32),
            pltpu.VMEM((NBLK, BLK), jnp.int32),
            pltpu.VMEM((CHUNK, 2), jnp.float32),
            pltpu.VMEM((CHUNK, 3), jnp.float32),
            pltpu.VMEM((CHUNK, 2), jnp.float32),
            pltpu.SemaphoreType.DMA,
        ],
    )
    return f(points_2d, camera_indices, pi3, pose, points_3d)


# speedup vs baseline: 7.7443x; 2.6714x over previous
import jax
import jax.numpy as jnp
from jax import lax
from jax.experimental import pallas as pl
from jax.experimental.pallas import tpu as pltpu
from jax.experimental.pallas import tpu_sc as plsc

N_IMGS = 2000
N_PTS = 500000
N_OBS = 2000000

NC = 2
NS = 16
NW = NC * NS
L = 16

CHUNK = 3200
BLK = 128
NBLK = CHUNK // BLK
NROWS = N_OBS // BLK
NCHUNKS = N_OBS // CHUNK
KMAX = -(-NCHUNKS // NW)


def _body(uv2d_hbm, ci_hbm, pi_hbm, pose_hbm, pts4_hbm, ouv_hbm,
          pose_v, ci_v, pi_v, uv_v, pts_v, ouv_v, sem):
    wid = lax.axis_index("s") * NC + lax.axis_index("c")
    pltpu.sync_copy(pose_hbm, pose_v)

    iota = lax.iota(jnp.int32, L)
    zero = jnp.zeros((L,), jnp.int32)
    one = jnp.full((L,), 1, jnp.int32)
    two = jnp.full((L,), 2, jnp.int32)

    def do_chunk(k, _):
        c = k * NW + wid

        @pl.when(c < NCHUNKS)
        def _():
            base = c * CHUNK
            rbase = c * NBLK
            pltpu.sync_copy(ci_hbm.at[pl.ds(base, CHUNK)], ci_v)
            pltpu.sync_copy(pi_hbm.at[pl.ds(rbase, NBLK)], pi_v)
            pltpu.sync_copy(uv2d_hbm.at[pl.ds(rbase, NBLK)],
                            uv_v.at[pl.ds(0, NBLK)])
            pltpu.sync_copy(uv2d_hbm.at[pl.ds(NROWS + rbase, NBLK)],
                            uv_v.at[pl.ds(NBLK, NBLK)])
            copies = [
                pltpu.async_copy(pts4_hbm.at[pi_v.at[j]],
                                 pts_v.at[pl.ds(j * BLK, BLK)], sem)
                for j in range(NBLK)
            ]
            for cp in copies:
                cp.wait()

            def grp(g, _):
                o = g * L + iota
                row = lax.shift_right_logical(o, 7)
                col = lax.bitwise_and(o, 127)
                ci = ci_v[pl.ds(g * L, L)]
                prow = lax.shift_right_logical(ci, 3)
                pbase = lax.shift_left(lax.bitwise_and(ci, 7), 4)
                cam = [plsc.load_gather(pose_v, [prow, pbase + j])
                       for j in range(10)]
                tx, ty, tz, qx, qy, qz, qw, fo, k1, k2 = cam
                px = plsc.load_gather(pts_v, [o, zero])
                py = plsc.load_gather(pts_v, [o, one])
                pz = plsc.load_gather(pts_v, [o, two])
                u2d = plsc.load_gather(uv_v, [row, col])
                v2d = plsc.load_gather(uv_v, [row + NBLK, col])

                nq = qx * qx + qy * qy + qz * qz + qw * qw
                c1x = qy * pz - qz * py
                c1y = qz * px - qx * pz
                c1z = qx * py - qy * px
                c2x = qy * c1z - qz * c1y
                c2y = qz * c1x - qx * c1z
                c2z = qx * c1y - qy * c1x
                s = 2.0 / nq
                rx = px + s * (qw * c1x + c2x) + tx
                ry = py + s * (qw * c1y + c2y) + ty
                rz = pz + s * (qw * c1z + c2z) + tz
                u = rx / rz
                v = ry / rz
                nr = u * u + v * v
                rad = 1.0 + k1 * nr + k2 * nr * nr
                rf = rad * fo
                plsc.store_scatter(ouv_v, [row, col], u * rf - u2d)
                plsc.store_scatter(ouv_v, [row + NBLK, col], v * rf - v2d)
                return 0

            lax.fori_loop(0, CHUNK // L, grp, 0)
            pltpu.sync_copy(ouv_v.at[pl.ds(0, NBLK)],
                            ouv_hbm.at[pl.ds(rbase, NBLK)])
            pltpu.sync_copy(ouv_v.at[pl.ds(NBLK, NBLK)],
                            ouv_hbm.at[pl.ds(NROWS + rbase, NBLK)])

        return 0

    lax.fori_loop(0, KMAX, do_chunk, 0)


def kernel(points_2d, camera_indices, point_indices, pose, points_3d):
    uv2d = jnp.concatenate(
        [points_2d[:, 0].reshape(NROWS, BLK),
         points_2d[:, 1].reshape(NROWS, BLK)], axis=0)
    pi2 = point_indices.reshape(NROWS, BLK)
    pose4 = jnp.pad(pose, ((0, 0), (0, 6))).reshape(N_IMGS // 8, BLK)
    pts4 = jnp.pad(points_3d, ((0, 0), (0, 1)))
    mesh = plsc.VectorSubcoreMesh(core_axis_name="c", subcore_axis_name="s")
    f = pl.kernel(
        _body,
        mesh=mesh,
        compiler_params=pltpu.CompilerParams(
            needs_layout_passes=False, use_tc_tiling_on_sc=False),
        out_type=jax.ShapeDtypeStruct((2 * NROWS, BLK), jnp.float32),
        scratch_types=[
            pltpu.VMEM((N_IMGS // 8, BLK), jnp.float32),
            pltpu.VMEM((CHUNK,), jnp.int32),
            pltpu.VMEM((NBLK, BLK), jnp.int32),
            pltpu.VMEM((2 * NBLK, BLK), jnp.float32),
            pltpu.VMEM((CHUNK, 4), jnp.float32),
            pltpu.VMEM((2 * NBLK, BLK), jnp.float32),
            pltpu.SemaphoreType.DMA,
        ],
    )
    ouv = f(uv2d, camera_indices, pi2, pose4, pts4)
    return jnp.stack([ouv[:NROWS].reshape(-1),
                      ouv[NROWS:].reshape(-1)], axis=-1)
